# baseline (speedup 1.0000x reference)
import functools

import jax
import jax.numpy as jnp
from jax import lax
from jax.experimental import pallas as pl
from jax.experimental.pallas import tpu as pltpu
from jax.experimental.pallas import tpu_sc as plsc

VOCAB = 100000
EMBED_DIM = 128
BATCH = 4096
SEQ = 200

_NUM_ROWS = BATCH * SEQ
_NW = 32
_ROWS_PER_W = _NUM_ROWS // _NW
_NB = 4
_CHUNK = 200
_N_CHUNKS = _ROWS_PER_W // _CHUNK


def _make_gather():
  mesh = plsc.VectorSubcoreMesh(core_axis_name="c", subcore_axis_name="s")

  @functools.partial(
      pl.kernel,
      out_type=jax.ShapeDtypeStruct((_NUM_ROWS, EMBED_DIM), jnp.float32),
      mesh=mesh,
      scratch_types=(
          [pltpu.VMEM((_ROWS_PER_W,), jnp.int32)]
          + [pltpu.VMEM((_CHUNK, EMBED_DIM), jnp.float32)] * _NB
          + [pltpu.SemaphoreType.DMA] * (2 * _NB)
      ),
  )
  def gather_kernel(table_hbm, idx_hbm, out_hbm, idx_v, *bufs):
    rows_v = bufs[:_NB]
    gsem = bufs[_NB:2 * _NB]
    osem = bufs[2 * _NB:]
    wid = lax.axis_index("s") * 2 + lax.axis_index("c")
    base = wid * _ROWS_PER_W

    pltpu.sync_copy(idx_hbm.at[pl.ds(base, _ROWS_PER_W)], idx_v)

    def start_gather(chunk, b):
      pltpu.async_copy(
          table_hbm.at[idx_v.at[pl.ds(chunk * _CHUNK, _CHUNK)]],
          rows_v[b], gsem[b])

    def wait_gather(chunk, b):
      pltpu.make_async_copy(
          table_hbm.at[idx_v.at[pl.ds(chunk * _CHUNK, _CHUNK)]],
          rows_v[b], gsem[b]).wait()

    def start_out(chunk, b):
      off = base + chunk * _CHUNK
      pltpu.async_copy(rows_v[b], out_hbm.at[pl.ds(off, _CHUNK)], osem[b])

    def wait_out(chunk, b):
      off = base + chunk * _CHUNK
      pltpu.make_async_copy(
          rows_v[b], out_hbm.at[pl.ds(off, _CHUNK)], osem[b]).wait()

    for b in range(_NB - 1):
      start_gather(b, b)

    @pl.loop(0, _N_CHUNKS, step=_NB)
    def _chunk_loop(i):
      for b in range(_NB):
        cur = i + b
        nxt = cur + _NB - 1
        @pl.when(nxt < _N_CHUNKS)
        def _():
          @pl.when(cur >= 1)
          def _():
            wait_out(cur - 1, (b - 1) % _NB)
          start_gather(nxt, (b - 1) % _NB)

        wait_gather(cur, b)
        start_out(cur, b)

    for c in range(_N_CHUNKS - _NB, _N_CHUNKS):
      wait_out(c, c % _NB)

  return gather_kernel


_gather = _make_gather()


@jax.jit
def kernel(input_ids, weight):
  idx = input_ids.reshape(_NUM_ROWS).astype(jnp.int32)
  out = _gather(weight, idx)
  return out.reshape(BATCH, SEQ, EMBED_DIM)
